# Initial kernel scaffold; baseline (speedup 1.0000x reference)
import jax
import jax.numpy as jnp
from jax.experimental import pallas as pl


def kernel(x, complexity, W_router, cg_w, cg_b):
    raise NotImplementedError("write your pallas kernel here")


# speedup vs baseline: 1.0071x; 1.0071x over previous
import functools

import jax
import jax.numpy as jnp
from jax.experimental import pallas as pl

_BS = 512


def _router_blk(x_ref, w_ref, gates_ref, idx_ref):
    logits = jax.lax.dot_general(
        w_ref[:], x_ref[:],
        (((1,), (1,)), ((), ())),
        preferred_element_type=jnp.float32,
    )
    m = jnp.max(logits, axis=0)
    s = jnp.sum(jnp.exp(logits - m[None, :]), axis=0)
    gates_ref[0, 0, :] = 1.0 / s
    idx_ref[0, 0, :] = jnp.argmax(logits, axis=0).astype(jnp.int32)


@functools.partial(jax.jit, static_argnames=())
def kernel(x, complexity, W_router, cg_w, cg_b):
    B, S, D = x.shape
    E = W_router.shape[0]
    n = (B * S) // _BS
    x2 = x.reshape(B * S, D)
    gates, idx = pl.pallas_call(
        _router_blk,
        grid=(n,),
        in_specs=[
            pl.BlockSpec((_BS, D), lambda i: (i, 0)),
            pl.BlockSpec((E, D), lambda i: (0, 0)),
        ],
        out_specs=[
            pl.BlockSpec((1, 1, _BS), lambda i: (i, 0, 0)),
            pl.BlockSpec((1, 1, _BS), lambda i: (i, 0, 0)),
        ],
        out_shape=[
            jax.ShapeDtypeStruct((n, 1, _BS), jnp.float32),
            jax.ShapeDtypeStruct((n, 1, _BS), jnp.int32),
        ],
    )(x2, W_router)
    return gates.reshape(B, S), idx.reshape(B, S)
